# Initial kernel scaffold; baseline (speedup 1.0000x reference)
import jax
import jax.numpy as jnp
from jax.experimental import pallas as pl


def kernel(user_ids, movie_ids, user_emb, movie_emb, W1, b1, W2, b2, W3, b3):
    raise NotImplementedError("write your pallas kernel here")


# speedup vs baseline: 2.3756x; 2.3756x over previous
import jax
import jax.numpy as jnp
from jax.experimental import pallas as pl
from jax.experimental.pallas import tpu as pltpu
from jax.experimental.pallas import tpu_sc as plsc


_GATHER_WINDOW = 128


def _sc_gather(user_emb, movie_emb, uids, mids):
    B = uids.shape[0]
    D = user_emb.shape[1]
    mesh = plsc.VectorSubcoreMesh(core_axis_name="core", subcore_axis_name="subcore")

    uids2 = uids.reshape(1, B)
    mids2 = mids.reshape(1, B)

    @pl.kernel(
        out_type=(
            jax.ShapeDtypeStruct((B, D), user_emb.dtype),
            jax.ShapeDtypeStruct((B, D), movie_emb.dtype),
        ),
        mesh=mesh,
    )
    def gather_kernel(ue_hbm, me_hbm, ui_hbm, mi_hbm, ou_hbm, om_hbm):
        def body(ui_vmem, mi_vmem, ou_vmem, om_vmem):
            pltpu.sync_copy(ue_hbm.at[ui_vmem.at[0]], ou_vmem)
            pltpu.sync_copy(me_hbm.at[mi_vmem.at[0]], om_vmem)

        pltpu.emit_pipeline(
            body,
            grid=(B // _GATHER_WINDOW,),
            in_specs=[
                pl.BlockSpec((1, _GATHER_WINDOW), lambda i: (0, i)),
                pl.BlockSpec((1, _GATHER_WINDOW), lambda i: (0, i)),
            ],
            out_specs=[
                pl.BlockSpec((_GATHER_WINDOW, D), lambda i: (i, 0)),
                pl.BlockSpec((_GATHER_WINDOW, D), lambda i: (i, 0)),
            ],
            core_axis_name=("core", "subcore"),
            dimension_semantics=(pltpu.PARALLEL,),
        )(ui_hbm, mi_hbm, ou_hbm, om_hbm)

    return gather_kernel(user_emb, movie_emb, uids2, mids2)


_MLP_BLOCK = 2048


def _mlp_body(u_ref, m_ref, w1u_ref, w1m_ref, b1_ref, w2_ref, b2_ref,
              w3_ref, b3_ref, o_ref):
    h = jnp.dot(u_ref[...], w1u_ref[...], preferred_element_type=jnp.float32)
    h = h + jnp.dot(m_ref[...], w1m_ref[...], preferred_element_type=jnp.float32)
    h = jnp.maximum(h + b1_ref[...], 0.0)
    h2 = jnp.dot(h, w2_ref[...], preferred_element_type=jnp.float32)
    h2 = jnp.maximum(h2 + b2_ref[...], 0.0)
    o_ref[...] = jnp.sum(h2 * w3_ref[...], axis=1) + b3_ref[0, 0]


def _mlp(u, m, W1, b1, W2, b2, W3, b3):
    B, D = u.shape
    w1u_t = W1[:, :D].T
    w1m_t = W1[:, D:].T
    w2_t = W2.T
    b1_2d = b1.reshape(1, -1)
    b2_2d = b2.reshape(1, -1)
    w3_2d = W3
    b3_2d = b3.reshape(1, 1)

    grid = (B // _MLP_BLOCK,)
    return pl.pallas_call(
        _mlp_body,
        grid=grid,
        in_specs=[
            pl.BlockSpec((_MLP_BLOCK, D), lambda i: (i, 0)),
            pl.BlockSpec((_MLP_BLOCK, D), lambda i: (i, 0)),
            pl.BlockSpec(w1u_t.shape, lambda i: (0, 0)),
            pl.BlockSpec(w1m_t.shape, lambda i: (0, 0)),
            pl.BlockSpec(b1_2d.shape, lambda i: (0, 0)),
            pl.BlockSpec(w2_t.shape, lambda i: (0, 0)),
            pl.BlockSpec(b2_2d.shape, lambda i: (0, 0)),
            pl.BlockSpec(w3_2d.shape, lambda i: (0, 0)),
            pl.BlockSpec(b3_2d.shape, lambda i: (0, 0)),
        ],
        out_specs=pl.BlockSpec((_MLP_BLOCK,), lambda i: (i,)),
        out_shape=jax.ShapeDtypeStruct((B,), jnp.float32),
    )(u, m, w1u_t, w1m_t, b1_2d, w2_t, b2_2d, w3_2d, b3_2d)


def kernel(user_ids, movie_ids, user_emb, movie_emb, W1, b1, W2, b2, W3, b3):
    u, m = _sc_gather(user_emb, movie_emb, user_ids, movie_ids)
    return _mlp(u, m, W1, b1, W2, b2, W3, b3)


# speedup vs baseline: 3.2513x; 1.3686x over previous
import jax
import jax.numpy as jnp
from jax.experimental import pallas as pl
from jax.experimental.pallas import tpu as pltpu
from jax.experimental.pallas import tpu_sc as plsc


_GATHER_WINDOW = 128


def _sc_gather(user_emb, movie_emb, uids, mids):
    B = uids.shape[0]
    D = user_emb.shape[1]
    mesh = plsc.VectorSubcoreMesh(core_axis_name="core", subcore_axis_name="subcore")

    uids2 = uids.reshape(1, B)
    mids2 = mids.reshape(1, B)

    @pl.kernel(
        out_type=(
            jax.ShapeDtypeStruct((B, D), user_emb.dtype),
            jax.ShapeDtypeStruct((B, D), movie_emb.dtype),
        ),
        mesh=mesh,
    )
    def gather_kernel(ue_hbm, me_hbm, ui_hbm, mi_hbm, ou_hbm, om_hbm):
        def body(ui_vmem, mi_vmem, ou_vmem, om_vmem):
            pltpu.sync_copy(ue_hbm.at[ui_vmem.at[0]], ou_vmem)
            pltpu.sync_copy(me_hbm.at[mi_vmem.at[0]], om_vmem)

        pltpu.emit_pipeline(
            body,
            grid=(B // _GATHER_WINDOW,),
            in_specs=[
                pl.BlockSpec((1, _GATHER_WINDOW), lambda i: (0, i)),
                pl.BlockSpec((1, _GATHER_WINDOW), lambda i: (0, i)),
            ],
            out_specs=[
                pl.BlockSpec((_GATHER_WINDOW, D), lambda i: (i, 0)),
                pl.BlockSpec((_GATHER_WINDOW, D), lambda i: (i, 0)),
            ],
            core_axis_name=("core", "subcore"),
            dimension_semantics=(pltpu.PARALLEL,),
        )(ui_hbm, mi_hbm, ou_hbm, om_hbm)

    return gather_kernel(user_emb, movie_emb, uids2, mids2)


_MLP_BLOCK = 2048


def _mlp_body(u_ref, m_ref, w1u_ref, w1m_ref, b1_ref, w2_ref, b2_ref,
              w3_ref, b3_ref, o_ref):
    h = jnp.dot(u_ref[...], w1u_ref[...], preferred_element_type=jnp.float32)
    h = h + jnp.dot(m_ref[...], w1m_ref[...], preferred_element_type=jnp.float32)
    h = jnp.maximum(h + b1_ref[...], 0.0)
    h2t = jax.lax.dot_general(w2_ref[...], h, (((1,), (1,)), ((), ())),
                              preferred_element_type=jnp.float32)
    h2t = jnp.maximum(h2t + b2_ref[...], 0.0)
    ot = jax.lax.dot_general(w3_ref[...], h2t, (((1,), (0,)), ((), ())),
                             preferred_element_type=jnp.float32)
    o_ref[...] = ot + b3_ref[0, 0]


def _mlp(u, m, W1, b1, W2, b2, W3, b3):
    B, D = u.shape
    w1u_t = W1[:, :D].T
    w1m_t = W1[:, D:].T
    b1_2d = b1.reshape(1, -1)
    b2_2d = b2.reshape(-1, 1)
    w3_2d = W3
    b3_2d = b3.reshape(1, 1)

    grid = (B // _MLP_BLOCK,)
    out_t = pl.pallas_call(
        _mlp_body,
        grid=grid,
        in_specs=[
            pl.BlockSpec((_MLP_BLOCK, D), lambda i: (i, 0)),
            pl.BlockSpec((_MLP_BLOCK, D), lambda i: (i, 0)),
            pl.BlockSpec(w1u_t.shape, lambda i: (0, 0)),
            pl.BlockSpec(w1m_t.shape, lambda i: (0, 0)),
            pl.BlockSpec(b1_2d.shape, lambda i: (0, 0)),
            pl.BlockSpec(W2.shape, lambda i: (0, 0)),
            pl.BlockSpec(b2_2d.shape, lambda i: (0, 0)),
            pl.BlockSpec(w3_2d.shape, lambda i: (0, 0)),
            pl.BlockSpec(b3_2d.shape, lambda i: (0, 0)),
        ],
        out_specs=pl.BlockSpec((1, _MLP_BLOCK), lambda i: (0, i)),
        out_shape=jax.ShapeDtypeStruct((1, B), jnp.float32),
    )(u, m, w1u_t, w1m_t, b1_2d, W2, b2_2d, w3_2d, b3_2d)
    return out_t.reshape(B)


def kernel(user_ids, movie_ids, user_emb, movie_emb, W1, b1, W2, b2, W3, b3):
    u, m = _sc_gather(user_emb, movie_emb, user_ids, movie_ids)
    return _mlp(u, m, W1, b1, W2, b2, W3, b3)
